# baseline (speedup 1.0000x reference)
import functools

import jax
import jax.numpy as jnp
from jax import lax
from jax.experimental import pallas as pl
from jax.experimental.pallas import tpu as pltpu
from jax.experimental.pallas import tpu_sc as plsc

C = 1000
CP = 1024
D = 256
B = 16384
BLK = 2048
NB = B // BLK

NH = 2
NG = 16
W = D // NG
HB = B // NH
RCH = 512
NCHB = HB // RCH
GRP = 16


def _offsets_body(label_ref, pred_ref, truth_ref, off_ref, cnt_ref):
    i = pl.program_id(0)
    pred = pred_ref[...]
    truth = truth_ref[...]
    labels = label_ref[0, 0, :]

    pn = jnp.sqrt(jnp.sum(pred * pred, axis=1, keepdims=True))
    p_ = pred / (pn + 1e-10)
    tn = jnp.sqrt(jnp.sum(truth * truth, axis=1, keepdims=True))
    t_ = truth / (tn + 1e-10)
    off_ref[...] = (p_ - t_) ** 2

    onehot = (labels[:, None] ==
              lax.broadcasted_iota(jnp.int32, (BLK, CP), 1)).astype(jnp.float32)
    ones = jnp.ones((BLK, 128), dtype=jnp.float32)
    cnt_part = lax.dot_general(onehot, ones, (((0,), (0,)), ((), ())),
                               preferred_element_type=jnp.float32)

    @pl.when(i == 0)
    def _init():
        cnt_ref[...] = cnt_part

    @pl.when(i > 0)
    def _acc():
        cnt_ref[...] += cnt_part


def _sc_segsum(off_hbm, lab_hbm, out_hbm, acc_v, buf_v, lab_v, sem_in, sem_lab):
    h = lax.axis_index("c")
    g = lax.axis_index("s")
    row0 = h * HB
    col0 = g * W

    zeros16 = jnp.zeros((16,), jnp.float32)
    iota16 = lax.iota(jnp.int32, 16)

    def zrow(r, _):
        acc_v[r, :] = zeros16
        return 0
    lax.fori_loop(0, CP, zrow, 0)

    cps = [None, None]
    lps = [None, None]
    cps[0] = pltpu.async_copy(
        off_hbm.at[pl.ds(row0, RCH), pl.ds(col0, W)], buf_v.at[0], sem_in)
    lps[0] = pltpu.async_copy(
        lab_hbm.at[pl.ds(row0, RCH)], lab_v.at[0], sem_lab)

    for ch in range(NCHB):
        b = ch % 2
        if ch + 1 < NCHB:
            nb = (ch + 1) % 2
            cps[nb] = pltpu.async_copy(
                off_hbm.at[pl.ds(row0 + (ch + 1) * RCH, RCH), pl.ds(col0, W)],
                buf_v.at[nb], sem_in)
            lps[nb] = pltpu.async_copy(
                lab_hbm.at[pl.ds(row0 + (ch + 1) * RCH, RCH)],
                lab_v.at[nb], sem_lab)
        cps[b].wait()
        lps[b].wait()

        def grp_body(gi, _):
            labs = lab_v[b, pl.ds(gi * GRP, GRP)]
            rows = jnp.broadcast_to(gi * GRP, (16,)) + iota16
            for j in range(W):
                colj = jnp.broadcast_to(jnp.int32(j), (16,))
                vals = plsc.load_gather(buf_v.at[b], [rows, colj])
                plsc.addupdate_scatter(acc_v, [labs, colj], vals)
            return 0
        lax.fori_loop(0, RCH // GRP, grp_body, 0)

    pltpu.sync_copy(acc_v, out_hbm.at[h, g])


_sc_segsum_call = functools.partial(
    pl.kernel,
    out_type=jax.ShapeDtypeStruct((NH, NG, CP, W), jnp.float32),
    mesh=plsc.VectorSubcoreMesh(core_axis_name="c", subcore_axis_name="s"),
    compiler_params=pltpu.CompilerParams(use_tc_tiling_on_sc=False,
                                         needs_layout_passes=False),
    scratch_types=[
        pltpu.VMEM((CP, W), jnp.float32),
        pltpu.VMEM((2, RCH, W), jnp.float32),
        pltpu.VMEM((2, RCH), jnp.int32),
        pltpu.SemaphoreType.DMA,
        pltpu.SemaphoreType.DMA,
    ],
)(_sc_segsum)


def _weights_body(part_ref, cnt_ref, mean_ref, w_ref):
    pieces = [part_ref[0, g] + part_ref[1, g] for g in range(NG)]
    sums = jnp.concatenate(pieces, axis=1)
    cnt = cnt_ref[:, 0:1]
    mean = sums / jnp.maximum(cnt, 1.0)
    mask = mean > 0.0
    big = jnp.where(mask, mean, jnp.inf)
    col_min = jnp.min(big, axis=1, keepdims=True)
    row_min = jnp.min(big, axis=0, keepdims=True)
    col_min = jnp.where(col_min < jnp.inf, col_min, 1.0)
    row_min = jnp.where(row_min < jnp.inf, row_min, 1.0)
    safe = jnp.where(mask, mean, 1.0)
    w1 = jnp.log(safe / row_min) + 1.0
    w2 = jnp.log(safe / col_min) + 1.0
    w = jnp.where(mask, w1 * w2, 1.0)
    mean_ref[...] = mean
    w_ref[...] = w


@jax.jit
def kernel(batch_pred, batch_truth, batch_label):
    labels3 = batch_label.reshape(NB, 1, BLK)
    offsets, cnts = pl.pallas_call(
        _offsets_body,
        grid=(NB,),
        in_specs=[
            pl.BlockSpec((1, 1, BLK), lambda i: (i, 0, 0)),
            pl.BlockSpec((BLK, D), lambda i: (i, 0)),
            pl.BlockSpec((BLK, D), lambda i: (i, 0)),
        ],
        out_specs=[
            pl.BlockSpec((BLK, D), lambda i: (i, 0)),
            pl.BlockSpec((CP, 128), lambda i: (0, 0)),
        ],
        out_shape=[
            jax.ShapeDtypeStruct((B, D), jnp.float32),
            jax.ShapeDtypeStruct((CP, 128), jnp.float32),
        ],
    )(labels3, batch_pred, batch_truth)

    parts = _sc_segsum_call(offsets, batch_label)

    mean_p, w_p = pl.pallas_call(
        _weights_body,
        out_shape=[
            jax.ShapeDtypeStruct((CP, D), jnp.float32),
            jax.ShapeDtypeStruct((CP, D), jnp.float32),
        ],
    )(parts, cnts)
    return (mean_p[:C], w_p[:C])


# speedup vs baseline: 1.3077x; 1.3077x over previous
import functools

import jax
import jax.numpy as jnp
from jax import lax
from jax.experimental import pallas as pl
from jax.experimental.pallas import tpu as pltpu
from jax.experimental.pallas import tpu_sc as plsc

C = 1000
CP = 1024
D = 256
B = 16384
BLK = 2048
NB = B // BLK

NH = 2
NG = 16
W = D // NG
HB = B // NH
RCH = 512
NCHB = HB // RCH
GRP = 16


def _offsets_body(label_ref, pred_ref, truth_ref, off_ref, cnt_ref):
    i = pl.program_id(0)
    pred = pred_ref[...]
    truth = truth_ref[...]
    labels = label_ref[0, 0, :]

    pn = jnp.sqrt(jnp.sum(pred * pred, axis=1, keepdims=True))
    p_ = pred / (pn + 1e-10)
    tn = jnp.sqrt(jnp.sum(truth * truth, axis=1, keepdims=True))
    t_ = truth / (tn + 1e-10)
    off_ref[...] = (p_ - t_) ** 2

    onehot = (labels[:, None] ==
              lax.broadcasted_iota(jnp.int32, (BLK, CP), 1)).astype(jnp.float32)
    ones = jnp.ones((BLK, 128), dtype=jnp.float32)
    cnt_part = lax.dot_general(onehot, ones, (((0,), (0,)), ((), ())),
                               preferred_element_type=jnp.float32)

    @pl.when(i == 0)
    def _init():
        cnt_ref[...] = cnt_part

    @pl.when(i > 0)
    def _acc():
        cnt_ref[...] += cnt_part


def _sc_segsum(off_hbm, lab_hbm, out_hbm, acc_v, buf_v, lab_v, sem_in, sem_lab):
    h = lax.axis_index("c")
    g = lax.axis_index("s")
    row0 = h * HB
    col0 = g * W

    zeros16 = jnp.zeros((16,), jnp.float32)
    iota16 = lax.iota(jnp.int32, 16)

    @plsc.parallel_loop(0, CP, GRP)
    def zrow(r):
        for rr in range(GRP):
            acc_v[r + rr, :] = zeros16

    cps = [None, None]
    lps = [None, None]
    cps[0] = pltpu.async_copy(
        off_hbm.at[pl.ds(row0, RCH), pl.ds(col0, W)], buf_v.at[0], sem_in)
    lps[0] = pltpu.async_copy(
        lab_hbm.at[pl.ds(row0, RCH)], lab_v.at[0], sem_lab)

    for ch in range(NCHB):
        b = ch % 2
        if ch + 1 < NCHB:
            nb = (ch + 1) % 2
            cps[nb] = pltpu.async_copy(
                off_hbm.at[pl.ds(row0 + (ch + 1) * RCH, RCH), pl.ds(col0, W)],
                buf_v.at[nb], sem_in)
            lps[nb] = pltpu.async_copy(
                lab_hbm.at[pl.ds(row0 + (ch + 1) * RCH, RCH)],
                lab_v.at[nb], sem_lab)
        cps[b].wait()
        lps[b].wait()

        @plsc.parallel_loop(0, RCH // GRP, 1, unroll=2)
        def grp_body(gi):
            labs = lab_v[b, pl.ds(gi * GRP, GRP)]
            rows = jnp.broadcast_to(gi * GRP, (16,)) + iota16
            cols = [jnp.broadcast_to(jnp.int32(j), (16,)) for j in range(W)]
            vals = [plsc.load_gather(buf_v.at[b], [rows, cols[j]])
                    for j in range(W)]
            for j in range(W):
                plsc.addupdate_scatter(acc_v, [labs, cols[j]], vals[j])

    pltpu.sync_copy(acc_v, out_hbm.at[h, g])


_sc_segsum_call = functools.partial(
    pl.kernel,
    out_type=jax.ShapeDtypeStruct((NH, NG, CP, W), jnp.float32),
    mesh=plsc.VectorSubcoreMesh(core_axis_name="c", subcore_axis_name="s"),
    compiler_params=pltpu.CompilerParams(use_tc_tiling_on_sc=False,
                                         needs_layout_passes=False),
    scratch_types=[
        pltpu.VMEM((CP, W), jnp.float32),
        pltpu.VMEM((2, RCH, W), jnp.float32),
        pltpu.VMEM((2, RCH), jnp.int32),
        pltpu.SemaphoreType.DMA,
        pltpu.SemaphoreType.DMA,
    ],
)(_sc_segsum)


def _weights_body(part_ref, cnt_ref, mean_ref, w_ref):
    pieces = [part_ref[0, g] + part_ref[1, g] for g in range(NG)]
    sums = jnp.concatenate(pieces, axis=1)
    cnt = cnt_ref[:, 0:1]
    mean = sums / jnp.maximum(cnt, 1.0)
    mask = mean > 0.0
    big = jnp.where(mask, mean, jnp.inf)
    col_min = jnp.min(big, axis=1, keepdims=True)
    row_min = jnp.min(big, axis=0, keepdims=True)
    col_min = jnp.where(col_min < jnp.inf, col_min, 1.0)
    row_min = jnp.where(row_min < jnp.inf, row_min, 1.0)
    safe = jnp.where(mask, mean, 1.0)
    w1 = jnp.log(safe / row_min) + 1.0
    w2 = jnp.log(safe / col_min) + 1.0
    w = jnp.where(mask, w1 * w2, 1.0)
    mean_ref[...] = mean
    w_ref[...] = w


@jax.jit
def kernel(batch_pred, batch_truth, batch_label):
    labels3 = batch_label.reshape(NB, 1, BLK)
    offsets, cnts = pl.pallas_call(
        _offsets_body,
        grid=(NB,),
        in_specs=[
            pl.BlockSpec((1, 1, BLK), lambda i: (i, 0, 0)),
            pl.BlockSpec((BLK, D), lambda i: (i, 0)),
            pl.BlockSpec((BLK, D), lambda i: (i, 0)),
        ],
        out_specs=[
            pl.BlockSpec((BLK, D), lambda i: (i, 0)),
            pl.BlockSpec((CP, 128), lambda i: (0, 0)),
        ],
        out_shape=[
            jax.ShapeDtypeStruct((B, D), jnp.float32),
            jax.ShapeDtypeStruct((CP, 128), jnp.float32),
        ],
    )(labels3, batch_pred, batch_truth)

    parts = _sc_segsum_call(offsets, batch_label)

    mean_p, w_p = pl.pallas_call(
        _weights_body,
        out_shape=[
            jax.ShapeDtypeStruct((CP, D), jnp.float32),
            jax.ShapeDtypeStruct((CP, D), jnp.float32),
        ],
    )(parts, cnts)
    return (mean_p[:C], w_p[:C])
